# baseline (speedup 1.0000x reference)
import functools

import jax
import jax.numpy as jnp
from jax import lax
from jax.experimental import pallas as pl
from jax.experimental.pallas import tpu as pltpu
from jax.experimental.pallas import tpu_sc as plsc

_MIX = 4
_GROUP = 32


def _target_body(out_ref):
    i = lax.broadcasted_iota(jnp.int32, out_ref.shape, 0)
    j = lax.broadcasted_iota(jnp.int32, out_ref.shape, 1)
    out_ref[...] = (i // _GROUP) * _GROUP + (i % _GROUP + j) % _GROUP


def _make_permute(B, T, C, dtype):
    S = T // _MIX
    mesh = plsc.VectorSubcoreMesh(core_axis_name="c", subcore_axis_name="s")
    n_workers = mesh.num_cores * mesh.num_subcores
    slabs = B * _MIX
    per_w = slabs // n_workers

    @functools.partial(
        pl.kernel,
        out_type=jax.ShapeDtypeStruct((B, T, C), dtype),
        mesh=mesh,
        scratch_types=[pltpu.SemaphoreType.DMA],
        compiler_params=pltpu.CompilerParams(use_tc_tiling_on_sc=False),
    )
    def permute(p_hbm, out_hbm, sem):
        wid = lax.axis_index("s") * mesh.num_cores + lax.axis_index("c")

        def issue(k, carry):
            slab = wid * per_w + k
            b = slab // _MIX
            q = slab % _MIX
            src_b = (b // _GROUP) * _GROUP + (b % _GROUP + q) % _GROUP
            pltpu.make_async_copy(
                p_hbm.at[src_b, pl.ds(q * S, S)],
                out_hbm.at[b, pl.ds(q * S, S)],
                sem,
            ).start()
            return carry

        lax.fori_loop(0, per_w, issue, 0, unroll=False)

        def drain(k, carry):
            pltpu.make_async_copy(
                p_hbm.at[0, pl.ds(0, S)],
                out_hbm.at[0, pl.ds(0, S)],
                sem,
            ).wait()
            return carry

        lax.fori_loop(0, per_w, drain, 0, unroll=False)

    return permute


def kernel(patches):
    B, T, C = patches.shape
    assert B % _GROUP == 0 and T % _MIX == 0

    target = pl.pallas_call(
        _target_body,
        out_shape=jax.ShapeDtypeStruct((B, _MIX), jnp.int32),
    )()

    out = _make_permute(B, T, C, patches.dtype)(patches)
    return (out, target)


# speedup vs baseline: 1.1699x; 1.1699x over previous
import functools

import jax
import jax.numpy as jnp
from jax import lax
from jax.experimental import pallas as pl
from jax.experimental.pallas import tpu as pltpu
from jax.experimental.pallas import tpu_sc as plsc

_MIX = 4
_GROUP = 32


def _target_body(out_ref):
    i = lax.broadcasted_iota(jnp.int32, out_ref.shape, 0)
    j = lax.broadcasted_iota(jnp.int32, out_ref.shape, 1)
    out_ref[...] = (i // _GROUP) * _GROUP + (i % _GROUP + j) % _GROUP


def _make_permute(B, T, C, dtype):
    S = T // _MIX
    mesh = plsc.VectorSubcoreMesh(core_axis_name="c", subcore_axis_name="s")
    n_workers = mesh.num_cores * mesh.num_subcores
    b_per_w = B // n_workers

    vecs_per_row = C // 16

    bounds = []
    for q in range(1, _MIX):
        t0 = (q * S) // 8 * 8
        bounds.append((t0, q * S - t0))
    interiors = []
    for q in range(_MIX):
        lo = -(-(q * S) // 8) * 8
        hi = ((q + 1) * S) // 8 * 8 if q < _MIX - 1 else T
        interiors.append((lo, hi - lo))

    @functools.partial(
        pl.kernel,
        out_type=jax.ShapeDtypeStruct((B, T, C), dtype),
        mesh=mesh,
        scratch_types=[
            pltpu.VMEM((8, C), dtype),
            pltpu.VMEM((8, C), dtype),
            pltpu.SemaphoreType.DMA,
            pltpu.SemaphoreType.DMA,
        ],
    )
    def permute(p_hbm, out_hbm, buf_hi, buf_lo, sem, sem_out):
        wid = lax.axis_index("s") * mesh.num_cores + lax.axis_index("c")
        b0 = wid * b_per_w

        def src_of(b, q):
            return (b // _GROUP) * _GROUP + (b % _GROUP + q) % _GROUP

        def issue(k, carry):
            b = b0 + k
            for q, (lo, sz) in enumerate(interiors):
                pltpu.make_async_copy(
                    p_hbm.at[src_of(b, q), pl.ds(lo, sz)],
                    out_hbm.at[b, pl.ds(lo, sz)],
                    sem,
                ).start()
            return carry

        lax.fori_loop(0, b_per_w, issue, 0, unroll=False)

        def fix(k, carry):
            b = b0 + k
            for q, (t0, r0) in enumerate(bounds):
                hi_cp = pltpu.make_async_copy(
                    p_hbm.at[src_of(b, q + 1), pl.ds(t0, 8)], buf_hi, sem_out
                )
                lo_cp = pltpu.make_async_copy(
                    p_hbm.at[src_of(b, q), pl.ds(t0, 8)], buf_lo, sem_out
                )
                hi_cp.start()
                lo_cp.start()
                hi_cp.wait()
                lo_cp.wait()
                for r in range(r0):
                    for v in range(vecs_per_row):
                        buf_hi[r, pl.ds(v * 16, 16)] = buf_lo[r, pl.ds(v * 16, 16)]
                out_cp = pltpu.make_async_copy(
                    buf_hi, out_hbm.at[b, pl.ds(t0, 8)], sem_out
                )
                out_cp.start()
                out_cp.wait()
            return carry

        lax.fori_loop(0, b_per_w, fix, 0, unroll=False)

        def drain(k, carry):
            for _, (lo, sz) in enumerate(interiors):
                pltpu.make_async_copy(
                    p_hbm.at[0, pl.ds(lo, sz)],
                    out_hbm.at[0, pl.ds(lo, sz)],
                    sem,
                ).wait()
            return carry

        lax.fori_loop(0, b_per_w, drain, 0, unroll=False)

    return permute


def kernel(patches):
    B, T, C = patches.shape
    assert B % _GROUP == 0 and T % _MIX == 0

    target = pl.pallas_call(
        _target_body,
        out_shape=jax.ShapeDtypeStruct((B, _MIX), jnp.int32),
    )()

    out = _make_permute(B, T, C, patches.dtype)(patches)
    return (out, target)


# speedup vs baseline: 13.4964x; 11.5363x over previous
import functools

import jax
import jax.numpy as jnp
from jax import lax
from jax.experimental import pallas as pl
from jax.experimental.pallas import tpu as pltpu
from jax.experimental.pallas import tpu_sc as plsc

_MIX = 4
_GROUP = 32


def _target_body(out_ref):
    i = lax.broadcasted_iota(jnp.int32, out_ref.shape, 0)
    j = lax.broadcasted_iota(jnp.int32, out_ref.shape, 1)
    out_ref[...] = (i // _GROUP) * _GROUP + (i % _GROUP + j) % _GROUP


def _make_permute(B, T, C, dtype):
    S = T // _MIX
    mesh = plsc.VectorSubcoreMesh(core_axis_name="c", subcore_axis_name="s")
    n_workers = mesh.num_cores * mesh.num_subcores
    b_per_w = B // n_workers

    vecs_per_row = C // 16

    bounds = []
    for q in range(1, _MIX):
        t0 = (q * S) // 8 * 8
        bounds.append((t0, q * S - t0))
    interiors = []
    for q in range(_MIX):
        lo = -(-(q * S) // 8) * 8
        hi = ((q + 1) * S) // 8 * 8
        interiors.append((lo, hi - lo))
    tail_lo = T // 8 * 8
    tail_sz = T - tail_lo

    max_rows = max(sz for _, sz in interiors)
    pieces = [(k, q) for k in range(b_per_w) for q in range(_MIX)]
    n_pieces = len(pieces)

    @functools.partial(
        pl.kernel,
        out_type=jax.ShapeDtypeStruct((B, T, C), dtype),
        mesh=mesh,
        scratch_types=[
            pltpu.VMEM((2, max_rows, C), dtype),
            pltpu.VMEM((8, C), dtype),
            pltpu.VMEM((8, C), dtype),
            pltpu.VMEM((tail_sz, C), dtype),
            pltpu.SemaphoreType.DMA,
            pltpu.SemaphoreType.DMA,
            pltpu.SemaphoreType.DMA,
            pltpu.SemaphoreType.DMA,
            pltpu.SemaphoreType.DMA,
        ],
    )
    def permute(p_hbm, out_hbm, bufs, buf_hi, buf_lo, buf_tail,
                sr0, sr1, sw0, sw1, sfix):
        wid = lax.axis_index("s") * mesh.num_cores + lax.axis_index("c")
        b0 = wid * b_per_w
        sem_r = (sr0, sr1)
        sem_w = (sw0, sw1)

        def src_of(b, q):
            return (b // _GROUP) * _GROUP + (b % _GROUP + q) % _GROUP

        def rd(i):
            k, q = pieces[i]
            lo, sz = interiors[q]
            return pltpu.make_async_copy(
                p_hbm.at[src_of(b0 + k, q), pl.ds(lo, sz)],
                bufs.at[i % 2, pl.ds(0, sz)],
                sem_r[i % 2],
            )

        def wr(i):
            k, q = pieces[i]
            lo, sz = interiors[q]
            return pltpu.make_async_copy(
                bufs.at[i % 2, pl.ds(0, sz)],
                out_hbm.at[b0 + k, pl.ds(lo, sz)],
                sem_w[i % 2],
            )

        rd(0).start()
        for i in range(n_pieces):
            if i >= 1:
                wr(i - 1).wait()
            if i + 1 < n_pieces:
                rd(i + 1).start()
            rd(i).wait()
            wr(i).start()
        wr(n_pieces - 1).wait()

        def fix(k, carry):
            b = b0 + k
            for q, (t0, r0) in enumerate(bounds):
                hi_cp = pltpu.make_async_copy(
                    p_hbm.at[src_of(b, q + 1), pl.ds(t0, 8)], buf_hi, sfix
                )
                lo_cp = pltpu.make_async_copy(
                    p_hbm.at[src_of(b, q), pl.ds(t0, 8)], buf_lo, sfix
                )
                hi_cp.start()
                lo_cp.start()
                hi_cp.wait()
                lo_cp.wait()
                for r in range(r0):
                    for v in range(vecs_per_row):
                        buf_hi[r, pl.ds(v * 16, 16)] = buf_lo[r, pl.ds(v * 16, 16)]
                out_cp = pltpu.make_async_copy(
                    buf_hi, out_hbm.at[b, pl.ds(t0, 8)], sfix
                )
                out_cp.start()
                out_cp.wait()
            t_in = pltpu.make_async_copy(
                p_hbm.at[src_of(b, _MIX - 1), pl.ds(tail_lo, tail_sz)],
                buf_tail,
                sfix,
            )
            t_in.start()
            t_in.wait()
            t_out = pltpu.make_async_copy(
                buf_tail, out_hbm.at[b, pl.ds(tail_lo, tail_sz)], sfix
            )
            t_out.start()
            t_out.wait()
            return carry

        lax.fori_loop(0, b_per_w, fix, 0, unroll=False)

    return permute


def kernel(patches):
    B, T, C = patches.shape
    assert B % _GROUP == 0 and T % _MIX == 0

    target = pl.pallas_call(
        _target_body,
        out_shape=jax.ShapeDtypeStruct((B, _MIX), jnp.int32),
    )()

    out = _make_permute(B, T, C, patches.dtype)(patches)
    return (out, target)


# speedup vs baseline: 14.2545x; 1.0562x over previous
import functools

import jax
import jax.numpy as jnp
from jax import lax
from jax.experimental import pallas as pl
from jax.experimental.pallas import tpu as pltpu
from jax.experimental.pallas import tpu_sc as plsc

_MIX = 4
_GROUP = 32


def _target_body(out_ref):
    i = lax.broadcasted_iota(jnp.int32, out_ref.shape, 0)
    j = lax.broadcasted_iota(jnp.int32, out_ref.shape, 1)
    out_ref[...] = (i // _GROUP) * _GROUP + (i % _GROUP + j) % _GROUP


def _make_permute(B, T, C, dtype):
    S = T // _MIX
    mesh = plsc.VectorSubcoreMesh(core_axis_name="c", subcore_axis_name="s")
    n_workers = mesh.num_cores * mesh.num_subcores
    b_per_w = B // n_workers

    vecs_per_row = C // 16

    bounds = []
    for q in range(1, _MIX):
        t0 = (q * S) // 8 * 8
        bounds.append((t0, q * S - t0))
    interiors = []
    for q in range(_MIX):
        lo = -(-(q * S) // 8) * 8
        hi = ((q + 1) * S) // 8 * 8
        interiors.append((lo, hi - lo))
    tail_lo = T // 8 * 8
    tail_sz = T - tail_lo

    max_rows = max(sz for _, sz in interiors)
    pieces = [(k, q) for k in range(b_per_w) for q in range(_MIX)]
    n_pieces = len(pieces)
    n_slots = 3
    n_bnd = len(bounds)

    @functools.partial(
        pl.kernel,
        out_type=jax.ShapeDtypeStruct((B, T, C), dtype),
        mesh=mesh,
        scratch_types=[
            pltpu.VMEM((n_slots, max_rows, C), dtype),
            pltpu.VMEM((n_slots, tail_sz, C), dtype),
            pltpu.SemaphoreType.DMA,
            pltpu.SemaphoreType.DMA,
            pltpu.SemaphoreType.DMA,
            pltpu.SemaphoreType.DMA,
            pltpu.SemaphoreType.DMA,
            pltpu.SemaphoreType.DMA,
        ],
    )
    def permute(p_hbm, out_hbm, bufs, tbufs, sr0, sr1, sr2, sw0, sw1, sw2):
        wid = lax.axis_index("s") * mesh.num_cores + lax.axis_index("c")
        b0 = wid * b_per_w
        sem_r = (sr0, sr1, sr2)
        sem_w = (sw0, sw1, sw2)

        def src_of(b, q):
            return (b // _GROUP) * _GROUP + (b % _GROUP + q) % _GROUP

        def rd(i):
            k, q = pieces[i]
            lo, sz = interiors[q]
            return pltpu.make_async_copy(
                p_hbm.at[src_of(b0 + k, q), pl.ds(lo, sz)],
                bufs.at[i % n_slots, pl.ds(0, sz)],
                sem_r[i % n_slots],
            )

        def wr(i):
            k, q = pieces[i]
            lo, sz = interiors[q]
            return pltpu.make_async_copy(
                bufs.at[i % n_slots, pl.ds(0, sz)],
                out_hbm.at[b0 + k, pl.ds(lo, sz)],
                sem_w[i % n_slots],
            )

        rd(0).start()
        rd(1).start()
        for i in range(n_pieces):
            rd(i).wait()
            wr(i).start()
            if i + 2 < n_pieces:
                if i >= 1:
                    wr(i - 1).wait()
                rd(i + 2).start()
        for i in range(n_pieces - n_slots, n_pieces):
            wr(i).wait()

        def fr(k):
            b = b0 + k
            slot = k % n_slots
            cps = []
            for j, (t0, r0) in enumerate(bounds):
                cps.append(pltpu.make_async_copy(
                    p_hbm.at[src_of(b, j + 1), pl.ds(t0, 8)],
                    bufs.at[slot, pl.ds(j * 16, 8)],
                    sem_r[slot],
                ))
                cps.append(pltpu.make_async_copy(
                    p_hbm.at[src_of(b, j), pl.ds(t0, 8)],
                    bufs.at[slot, pl.ds(j * 16 + 8, 8)],
                    sem_r[slot],
                ))
            cps.append(pltpu.make_async_copy(
                p_hbm.at[src_of(b, _MIX - 1), pl.ds(tail_lo, tail_sz)],
                tbufs.at[slot],
                sem_r[slot],
            ))
            return cps

        def fmerge(k):
            slot = k % n_slots
            for j, (t0, r0) in enumerate(bounds):
                for r in range(r0):
                    for v in range(vecs_per_row):
                        bufs[slot, j * 16 + r, pl.ds(v * 16, 16)] = (
                            bufs[slot, j * 16 + 8 + r, pl.ds(v * 16, 16)]
                        )

        def fw(k):
            b = b0 + k
            slot = k % n_slots
            cps = []
            for j, (t0, r0) in enumerate(bounds):
                cps.append(pltpu.make_async_copy(
                    bufs.at[slot, pl.ds(j * 16, 8)],
                    out_hbm.at[b, pl.ds(t0, 8)],
                    sem_w[slot],
                ))
            cps.append(pltpu.make_async_copy(
                tbufs.at[slot], out_hbm.at[b, pl.ds(tail_lo, tail_sz)],
                sem_w[slot],
            ))
            return cps

        for cp in fr(0):
            cp.start()
        for cp in fr(1):
            cp.start()
        for k in range(b_per_w):
            for cp in fr(k):
                cp.wait()
            fmerge(k)
            for cp in fw(k):
                cp.start()
            if k + 2 < b_per_w:
                if k >= 1:
                    for cp in fw(k - 1):
                        cp.wait()
                for cp in fr(k + 2):
                    cp.start()
        for k in range(max(0, b_per_w - n_slots), b_per_w):
            for cp in fw(k):
                cp.wait()

    return permute


def kernel(patches):
    B, T, C = patches.shape
    assert B % _GROUP == 0 and T % _MIX == 0

    target = pl.pallas_call(
        _target_body,
        out_shape=jax.ShapeDtypeStruct((B, _MIX), jnp.int32),
    )()

    out = _make_permute(B, T, C, patches.dtype)(patches)
    return (out, target)


# speedup vs baseline: 39.5506x; 2.7746x over previous
import functools

import jax
import jax.numpy as jnp
from jax import lax
from jax.experimental import pallas as pl
from jax.experimental.pallas import tpu as pltpu
from jax.experimental.pallas import tpu_sc as plsc

_MIX = 4
_GROUP = 32


def _target_body(out_ref):
    i = lax.broadcasted_iota(jnp.int32, out_ref.shape, 0)
    j = lax.broadcasted_iota(jnp.int32, out_ref.shape, 1)
    out_ref[...] = (i // _GROUP) * _GROUP + (i % _GROUP + j) % _GROUP


def _make_permute(B, T, C, dtype):
    S = T // _MIX
    R = T * B
    mesh = plsc.VectorSubcoreMesh(core_axis_name="c", subcore_axis_name="s")
    n_workers = mesh.num_cores * mesh.num_subcores
    rows_per_w = R // n_workers
    chunk = 56
    n_chunks = rows_per_w // chunk
    n_slots = 2

    @functools.partial(
        pl.kernel,
        out_type=jax.ShapeDtypeStruct((R, C), dtype),
        mesh=mesh,
        scratch_types=[
            pltpu.VMEM((rows_per_w,), jnp.int32),
            pltpu.VMEM((n_slots, chunk, C), dtype),
            pltpu.SemaphoreType.DMA,
            pltpu.SemaphoreType.DMA,
            pltpu.SemaphoreType.DMA,
            pltpu.SemaphoreType.DMA,
        ],
    )
    def permute(p_hbm, out_hbm, idx, bufs, sg0, sg1, ss0, ss1):
        wid = lax.axis_index("s") * mesh.num_cores + lax.axis_index("c")
        r0 = wid * rows_per_w
        sem_g = (sg0, sg1)
        sem_s = (ss0, ss1)

        lane = lax.iota(jnp.int32, 16)
        groups_per_b = B // _GROUP
        u0 = r0 // _GROUP

        def gen_idx(j, carry):
            unit = u0 + j // 2
            half = j % 2
            t = unit // groups_per_b
            g = unit % groups_per_b
            q = t // S
            base = t * B + g * _GROUP
            rot = (lane + (half * 16 + q)) & (_GROUP - 1)
            idx[pl.ds(j * 16, 16)] = base + rot
            return carry

        lax.fori_loop(0, rows_per_w // 16, gen_idx, 0, unroll=False)

        def gather(i):
            return pltpu.make_async_copy(
                p_hbm.at[idx.at[pl.ds(i * chunk, chunk)]],
                bufs.at[i % n_slots],
                sem_g[i % n_slots],
            )

        def scatter(i):
            return pltpu.make_async_copy(
                bufs.at[i % n_slots],
                out_hbm.at[pl.ds(r0 + i * chunk, chunk)],
                sem_s[i % n_slots],
            )

        gather(0).start()
        for i in range(n_chunks):
            gather(i).wait()
            scatter(i).start()
            if i + 1 < n_chunks:
                if i >= 1:
                    scatter(i - 1).wait()
                gather(i + 1).start()
        scatter(n_chunks - 2).wait()
        scatter(n_chunks - 1).wait()

    return permute


def kernel(patches):
    B, T, C = patches.shape
    assert B % _GROUP == 0 and T % _MIX == 0

    target = pl.pallas_call(
        _target_body,
        out_shape=jax.ShapeDtypeStruct((B, _MIX), jnp.int32),
    )()

    pt = jnp.transpose(patches, (1, 0, 2)).reshape(T * B, C)
    out2d = _make_permute(B, T, C, patches.dtype)(pt)
    out = jnp.transpose(out2d.reshape(T, B, C), (1, 0, 2))
    return (out, target)


# speedup vs baseline: 39.9321x; 1.0096x over previous
import functools

import jax
import jax.numpy as jnp
from jax import lax
from jax.experimental import pallas as pl
from jax.experimental.pallas import tpu as pltpu
from jax.experimental.pallas import tpu_sc as plsc

_MIX = 4
_GROUP = 32


def _target_body(out_ref):
    i = lax.broadcasted_iota(jnp.int32, out_ref.shape, 0)
    j = lax.broadcasted_iota(jnp.int32, out_ref.shape, 1)
    out_ref[...] = (i // _GROUP) * _GROUP + (i % _GROUP + j) % _GROUP


def _make_permute(B, T, C, dtype):
    S = T // _MIX
    R = T * B
    mesh = plsc.VectorSubcoreMesh(core_axis_name="c", subcore_axis_name="s")
    n_workers = mesh.num_cores * mesh.num_subcores
    rows_per_w = R // n_workers
    chunk = 32
    n_chunks = rows_per_w // chunk
    n_slots = 4

    @functools.partial(
        pl.kernel,
        out_type=jax.ShapeDtypeStruct((R, C), dtype),
        mesh=mesh,
        scratch_types=[
            pltpu.VMEM((rows_per_w,), jnp.int32),
            pltpu.VMEM((n_slots, chunk, C), dtype),
            [pltpu.SemaphoreType.DMA] * n_slots,
            [pltpu.SemaphoreType.DMA] * n_slots,
        ],
    )
    def permute(p_hbm, out_hbm, idx, bufs, sem_g, sem_s):
        wid = lax.axis_index("s") * mesh.num_cores + lax.axis_index("c")
        r0 = wid * rows_per_w

        lane = lax.iota(jnp.int32, 16)
        groups_per_b = B // _GROUP
        u0 = r0 // _GROUP

        def gen_idx(j, carry):
            unit = u0 + j // 2
            half = j % 2
            t = unit // groups_per_b
            g = unit % groups_per_b
            q = t // S
            base = t * B + g * _GROUP
            rot = (lane + (half * 16 + q)) & (_GROUP - 1)
            idx[pl.ds(j * 16, 16)] = base + rot
            return carry

        lax.fori_loop(0, rows_per_w // 16, gen_idx, 0, unroll=False)

        def gather(i):
            return pltpu.make_async_copy(
                p_hbm.at[idx.at[pl.ds(i * chunk, chunk)]],
                bufs.at[i % n_slots],
                sem_g[i % n_slots],
            )

        def scatter(i):
            return pltpu.make_async_copy(
                bufs.at[i % n_slots],
                out_hbm.at[pl.ds(r0 + i * chunk, chunk)],
                sem_s[i % n_slots],
            )

        for i in range(min(n_slots - 1, n_chunks)):
            gather(i).start()
        for i in range(n_chunks):
            gather(i).wait()
            scatter(i).start()
            if i + n_slots - 1 < n_chunks:
                if i >= 1:
                    scatter(i - 1).wait()
                gather(i + n_slots - 1).start()
        for i in range(max(0, n_chunks - n_slots), n_chunks):
            scatter(i).wait()

    return permute


def kernel(patches):
    B, T, C = patches.shape
    assert B % _GROUP == 0 and T % _MIX == 0

    target = pl.pallas_call(
        _target_body,
        out_shape=jax.ShapeDtypeStruct((B, _MIX), jnp.int32),
    )()

    pt = jnp.transpose(patches, (1, 0, 2)).reshape(T * B, C)
    out2d = _make_permute(B, T, C, patches.dtype)(pt)
    out = jnp.transpose(out2d.reshape(T, B, C), (1, 0, 2))
    return (out, target)
